# Initial kernel scaffold; baseline (speedup 1.0000x reference)
import jax
import jax.numpy as jnp
from jax.experimental import pallas as pl


def kernel(a_feature, b_feature, edge_index):
    raise NotImplementedError("write your pallas kernel here")


# speedup vs baseline: 24.0770x; 24.0770x over previous
import functools

import jax
import jax.numpy as jnp
from jax import lax
from jax.experimental import pallas as pl
from jax.experimental.pallas import tpu as pltpu
from jax.experimental.pallas import tpu_sc as plsc

N = 50000
E = 800000
D = 64
EPS = 1e-08

NW = 32
TPS = 16
EPT1 = E // TPS
CH = 125
ROWS2 = E // CH
RPT = ROWS2 // TPS
NPAD = 50048
APT = NPAD // TPS
RCHUNK = 40
NB = 10
NBLK = N // NB
EBLK = ROWS2 // NB

_mesh = plsc.VectorSubcoreMesh(core_axis_name="c", subcore_axis_name="s")
_sc_params = pltpu.CompilerParams(needs_layout_passes=False,
                                  use_tc_tiling_on_sc=False)


@functools.partial(
    pl.kernel,
    out_type=jax.ShapeDtypeStruct((NW * N,), jnp.float32),
    mesh=_mesh,
    compiler_params=_sc_params,
    scratch_types=[
        pltpu.VMEM((EPT1,), jnp.int32),
        pltpu.VMEM((N,), jnp.float32),
    ],
)
def _deg_kernel(edges_hbm, zeros_hbm, out_hbm, idx_v, hist_v):
    c = lax.axis_index("c")
    s = lax.axis_index("s")
    pltpu.sync_copy(edges_hbm.at[pl.ds(c * E + s * EPT1, EPT1)], idx_v)
    pltpu.sync_copy(zeros_hbm, hist_v)
    ones = jnp.full((16,), 1.0, dtype=jnp.float32)

    def body(i, carry):
        iv = idx_v[pl.ds(i * 16, 16)]
        plsc.addupdate_scatter(hist_v, [iv], ones)
        return carry

    lax.fori_loop(0, EPT1 // 16, body, 0)
    pltpu.sync_copy(hist_v, out_hbm.at[pl.ds((c * TPS + s) * N, N)])


def _scale_body(dp_ref, a_ref, b_ref, src_ref, dst_ref,
                ta_ref, tb_ref, so_ref, do_ref):
    dp = dp_ref[0]
    deg_a = jnp.sum(dp[0:TPS, :], axis=0)
    deg_b = jnp.sum(dp[TPS:NW, :], axis=0)
    inv_a = 1.0 / (jnp.sqrt(deg_a) + EPS)
    inv_b = 1.0 / (jnp.sqrt(deg_b) + EPS)
    a = a_ref[...]
    b = b_ref[...]
    ta_ref[0] = a[:, 0:32] * inv_a[:, None]
    ta_ref[1] = a[:, 32:64] * inv_a[:, None]
    tb_ref[0] = b[:, 0:32] * inv_b[:, None]
    tb_ref[1] = b[:, 32:64] * inv_b[:, None]
    so_ref[0] = src_ref[...]
    so_ref[1] = src_ref[...] + N
    do_ref[0] = dst_ref[...]
    do_ref[1] = dst_ref[...] + N


_scale_call = pl.pallas_call(
    _scale_body,
    grid=(NB,),
    in_specs=[
        pl.BlockSpec((1, NW, NBLK), lambda i: (i, 0, 0)),
        pl.BlockSpec((NBLK, D), lambda i: (i, 0)),
        pl.BlockSpec((NBLK, D), lambda i: (i, 0)),
        pl.BlockSpec((EBLK, CH), lambda i: (i, 0)),
        pl.BlockSpec((EBLK, CH), lambda i: (i, 0)),
    ],
    out_specs=[
        pl.BlockSpec((2, NBLK, 32), lambda i: (0, i, 0)),
        pl.BlockSpec((2, NBLK, 32), lambda i: (0, i, 0)),
        pl.BlockSpec((2, EBLK, CH), lambda i: (0, i, 0)),
        pl.BlockSpec((2, EBLK, CH), lambda i: (0, i, 0)),
    ],
    out_shape=[
        jax.ShapeDtypeStruct((2, N, 32), jnp.float32),
        jax.ShapeDtypeStruct((2, N, 32), jnp.float32),
        jax.ShapeDtypeStruct((2, ROWS2, CH), jnp.int32),
        jax.ShapeDtypeStruct((2, ROWS2, CH), jnp.int32),
    ],
)


@functools.partial(
    pl.kernel,
    out_type=(
        jax.ShapeDtypeStruct((2, NPAD, 32), jnp.float32),
        jax.ShapeDtypeStruct((2, NPAD, 32), jnp.float32),
    ),
    mesh=_mesh,
    compiler_params=_sc_params,
    scratch_types=[
        pltpu.VMEM_SHARED((NPAD, 32), jnp.float32),
        pltpu.VMEM((RCHUNK, CH), jnp.int32),
        pltpu.VMEM((RCHUNK, CH), jnp.int32),
        pltpu.VMEM((CH, 32), jnp.float32),
    ],
)
def _spmm_kernel(tab_a, tab_b, src2, dst2, src_ofs, dst_ofs, zeros_hbm,
                 sa_out, sb_out, acc, gidx, sidx, rows):
    c = lax.axis_index("c")
    s = lax.axis_index("s")
    row0 = s * RPT
    acc0 = s * APT

    def phase(tab_hbm, gofs_hbm, sraw_hbm, out_hbm):
        pltpu.sync_copy(zeros_hbm, acc.at[pl.ds(acc0, APT)])
        plsc.subcore_barrier()

        def outer(t, carry):
            r0 = row0 + t * RCHUNK
            pltpu.sync_copy(gofs_hbm.at[c, pl.ds(r0, RCHUNK)], gidx)
            pltpu.sync_copy(sraw_hbm.at[pl.ds(r0, RCHUNK)], sidx)

            def body(j, carry2):
                pltpu.sync_copy(tab_hbm.at[gidx.at[j]], rows)
                pltpu.sync_copy(rows, acc.at[sidx.at[j]], add=True)
                return carry2

            return lax.fori_loop(0, RCHUNK, body, carry)

        lax.fori_loop(0, RPT // RCHUNK, outer, 0)
        plsc.subcore_barrier()
        pltpu.sync_copy(acc.at[pl.ds(acc0, APT)],
                        out_hbm.at[c, pl.ds(acc0, APT)])
        plsc.subcore_barrier()

    phase(tab_b, dst_ofs, src2, sa_out)
    phase(tab_a, src_ofs, dst2, sb_out)


def _combine_body(dp_ref, sa_ref, sb_ref, a_ref, b_ref, oa_ref, ob_ref):
    dp = dp_ref[0]
    deg_a = jnp.sum(dp[0:TPS, :], axis=0)
    deg_b = jnp.sum(dp[TPS:NW, :], axis=0)
    inv_a = 1.0 / (jnp.sqrt(deg_a) + EPS)
    inv_b = 1.0 / (jnp.sqrt(deg_b) + EPS)
    sa = jnp.concatenate([sa_ref[0], sa_ref[1]], axis=1)
    sb = jnp.concatenate([sb_ref[0], sb_ref[1]], axis=1)
    oa_ref[...] = 0.5 * (a_ref[...] + inv_a[:, None] * sa)
    ob_ref[...] = 0.5 * (b_ref[...] + inv_b[:, None] * sb)


_combine_call = pl.pallas_call(
    _combine_body,
    grid=(NB,),
    in_specs=[
        pl.BlockSpec((1, NW, NBLK), lambda i: (i, 0, 0)),
        pl.BlockSpec((2, NBLK, 32), lambda i: (0, i, 0)),
        pl.BlockSpec((2, NBLK, 32), lambda i: (0, i, 0)),
        pl.BlockSpec((NBLK, D), lambda i: (i, 0)),
        pl.BlockSpec((NBLK, D), lambda i: (i, 0)),
    ],
    out_specs=[
        pl.BlockSpec((NBLK, D), lambda i: (i, 0)),
        pl.BlockSpec((NBLK, D), lambda i: (i, 0)),
    ],
    out_shape=[
        jax.ShapeDtypeStruct((N, D), jnp.float32),
        jax.ShapeDtypeStruct((N, D), jnp.float32),
    ],
)


@jax.jit
def kernel(a_feature, b_feature, edge_index):
    ei = edge_index.astype(jnp.int32)
    src2 = ei[0].reshape(ROWS2, CH)
    dst2 = ei[1].reshape(ROWS2, CH)

    zeros_hist = jnp.zeros((N,), jnp.float32)
    deg_parts = (_deg_kernel(ei.reshape(2 * E), zeros_hist)
                 .reshape(NW, NB, NBLK).transpose(1, 0, 2))

    tab_a, tab_b, src_ofs, dst_ofs = _scale_call(
        deg_parts, a_feature, b_feature, src2, dst2)

    zeros_feat = jnp.zeros((APT, 32), jnp.float32)
    s_a, s_b = _spmm_kernel(
        tab_a.reshape(2 * N, 32), tab_b.reshape(2 * N, 32),
        src2, dst2, src_ofs, dst_ofs, zeros_feat)

    out_a, out_b = _combine_call(
        deg_parts, s_a[:, :N, :], s_b[:, :N, :], a_feature, b_feature)
    return out_a, out_b


# speedup vs baseline: 40.8197x; 1.6954x over previous
import functools

import jax
import jax.numpy as jnp
from jax import lax
from jax.experimental import pallas as pl
from jax.experimental.pallas import tpu as pltpu
from jax.experimental.pallas import tpu_sc as plsc

N = 50000
E = 800000
D = 64
EPS = 1e-08

NW = 32
TPS = 16
EPT1 = E // TPS
CH = 125
ROWS2 = E // CH
RPT = ROWS2 // TPS
NPAD = 50048
APT = NPAD // TPS
RCHUNK = 40
NBUF = 4
NB = 10
NBLK = N // NB
EBLK = ROWS2 // NB

_mesh = plsc.VectorSubcoreMesh(core_axis_name="c", subcore_axis_name="s")
_sc_params = pltpu.CompilerParams(needs_layout_passes=False,
                                  use_tc_tiling_on_sc=False)


@functools.partial(
    pl.kernel,
    out_type=jax.ShapeDtypeStruct((NW * N,), jnp.float32),
    mesh=_mesh,
    compiler_params=_sc_params,
    scratch_types=[
        pltpu.VMEM((EPT1,), jnp.int32),
        pltpu.VMEM((N,), jnp.float32),
    ],
)
def _deg_kernel(edges_hbm, zeros_hbm, out_hbm, idx_v, hist_v):
    c = lax.axis_index("c")
    s = lax.axis_index("s")
    pltpu.sync_copy(edges_hbm.at[pl.ds(c * E + s * EPT1, EPT1)], idx_v)
    pltpu.sync_copy(zeros_hbm, hist_v)
    ones = jnp.full((16,), 1.0, dtype=jnp.float32)

    def body(i, carry):
        iv = idx_v[pl.ds(i * 16, 16)]
        plsc.addupdate_scatter(hist_v, [iv], ones)
        return carry

    lax.fori_loop(0, EPT1 // 16, body, 0)
    pltpu.sync_copy(hist_v, out_hbm.at[pl.ds((c * TPS + s) * N, N)])


def _scale_body(dp_ref, a_ref, b_ref, src_ref, dst_ref,
                ta_ref, tb_ref, so_ref, do_ref):
    dp = dp_ref[0]
    deg_a = jnp.sum(dp[0:TPS, :], axis=0)
    deg_b = jnp.sum(dp[TPS:NW, :], axis=0)
    inv_a = 1.0 / (jnp.sqrt(deg_a) + EPS)
    inv_b = 1.0 / (jnp.sqrt(deg_b) + EPS)
    a = a_ref[...]
    b = b_ref[...]
    ta_ref[0] = a[:, 0:32] * inv_a[:, None]
    ta_ref[1] = a[:, 32:64] * inv_a[:, None]
    tb_ref[0] = b[:, 0:32] * inv_b[:, None]
    tb_ref[1] = b[:, 32:64] * inv_b[:, None]
    so_ref[0] = src_ref[...]
    so_ref[1] = src_ref[...] + N
    do_ref[0] = dst_ref[...]
    do_ref[1] = dst_ref[...] + N


_scale_call = pl.pallas_call(
    _scale_body,
    grid=(NB,),
    in_specs=[
        pl.BlockSpec((1, NW, NBLK), lambda i: (i, 0, 0)),
        pl.BlockSpec((NBLK, D), lambda i: (i, 0)),
        pl.BlockSpec((NBLK, D), lambda i: (i, 0)),
        pl.BlockSpec((EBLK, CH), lambda i: (i, 0)),
        pl.BlockSpec((EBLK, CH), lambda i: (i, 0)),
    ],
    out_specs=[
        pl.BlockSpec((2, NBLK, 32), lambda i: (0, i, 0)),
        pl.BlockSpec((2, NBLK, 32), lambda i: (0, i, 0)),
        pl.BlockSpec((2, EBLK, CH), lambda i: (0, i, 0)),
        pl.BlockSpec((2, EBLK, CH), lambda i: (0, i, 0)),
    ],
    out_shape=[
        jax.ShapeDtypeStruct((2, N, 32), jnp.float32),
        jax.ShapeDtypeStruct((2, N, 32), jnp.float32),
        jax.ShapeDtypeStruct((2, ROWS2, CH), jnp.int32),
        jax.ShapeDtypeStruct((2, ROWS2, CH), jnp.int32),
    ],
)


@functools.partial(
    pl.kernel,
    out_type=(
        jax.ShapeDtypeStruct((2, NPAD, 32), jnp.float32),
        jax.ShapeDtypeStruct((2, NPAD, 32), jnp.float32),
    ),
    mesh=_mesh,
    compiler_params=_sc_params,
    scratch_types=[
        pltpu.VMEM_SHARED((NPAD, 32), jnp.float32),
        pltpu.VMEM((RCHUNK, CH), jnp.int32),
        pltpu.VMEM((RCHUNK, CH), jnp.int32),
        [pltpu.VMEM((CH, 32), jnp.float32) for _ in range(NBUF)],
        [pltpu.SemaphoreType.DMA for _ in range(NBUF)],
        [pltpu.SemaphoreType.DMA for _ in range(NBUF)],
    ],
)
def _spmm_kernel(tab_a, tab_b, src2, dst2, src_ofs, dst_ofs, zeros_hbm,
                 sa_out, sb_out, acc, gidx, sidx, rows, gsem, ssem):
    c = lax.axis_index("c")
    s = lax.axis_index("s")
    row0 = s * RPT
    acc0 = s * APT

    def phase(tab_hbm, gofs_hbm, sraw_hbm, out_hbm):
        pltpu.sync_copy(zeros_hbm, acc.at[pl.ds(acc0, APT)])
        plsc.subcore_barrier()

        def outer(t, carry):
            r0 = row0 + t * RCHUNK
            pltpu.sync_copy(gofs_hbm.at[c, pl.ds(r0, RCHUNK)], gidx)
            pltpu.sync_copy(sraw_hbm.at[pl.ds(r0, RCHUNK)], sidx)

            def passes(q, carry2):
                j0 = q * NBUF
                for b in range(NBUF):
                    @pl.when(q > 0)
                    def _():
                        pltpu.make_async_copy(
                            rows[b], acc.at[sidx.at[0]], ssem[b]).wait()
                    pltpu.async_copy(
                        tab_hbm.at[gidx.at[j0 + b]], rows[b], gsem[b])
                for b in range(NBUF):
                    pltpu.make_async_copy(
                        tab_hbm.at[gidx.at[0]], rows[b], gsem[b]).wait()
                    pltpu.async_copy(
                        rows[b], acc.at[sidx.at[j0 + b]], ssem[b], add=True)
                return carry2

            lax.fori_loop(0, RCHUNK // NBUF, passes, carry)
            for b in range(NBUF):
                pltpu.make_async_copy(
                    rows[b], acc.at[sidx.at[0]], ssem[b]).wait()
            return carry

        lax.fori_loop(0, RPT // RCHUNK, outer, 0)
        plsc.subcore_barrier()
        pltpu.sync_copy(acc.at[pl.ds(acc0, APT)],
                        out_hbm.at[c, pl.ds(acc0, APT)])
        plsc.subcore_barrier()

    phase(tab_b, dst_ofs, src2, sa_out)
    phase(tab_a, src_ofs, dst2, sb_out)


def _combine_body(dp_ref, sa_ref, sb_ref, a_ref, b_ref, oa_ref, ob_ref):
    dp = dp_ref[0]
    deg_a = jnp.sum(dp[0:TPS, :], axis=0)
    deg_b = jnp.sum(dp[TPS:NW, :], axis=0)
    inv_a = 1.0 / (jnp.sqrt(deg_a) + EPS)
    inv_b = 1.0 / (jnp.sqrt(deg_b) + EPS)
    sa = jnp.concatenate([sa_ref[0], sa_ref[1]], axis=1)
    sb = jnp.concatenate([sb_ref[0], sb_ref[1]], axis=1)
    oa_ref[...] = 0.5 * (a_ref[...] + inv_a[:, None] * sa)
    ob_ref[...] = 0.5 * (b_ref[...] + inv_b[:, None] * sb)


_combine_call = pl.pallas_call(
    _combine_body,
    grid=(NB,),
    in_specs=[
        pl.BlockSpec((1, NW, NBLK), lambda i: (i, 0, 0)),
        pl.BlockSpec((2, NBLK, 32), lambda i: (0, i, 0)),
        pl.BlockSpec((2, NBLK, 32), lambda i: (0, i, 0)),
        pl.BlockSpec((NBLK, D), lambda i: (i, 0)),
        pl.BlockSpec((NBLK, D), lambda i: (i, 0)),
    ],
    out_specs=[
        pl.BlockSpec((NBLK, D), lambda i: (i, 0)),
        pl.BlockSpec((NBLK, D), lambda i: (i, 0)),
    ],
    out_shape=[
        jax.ShapeDtypeStruct((N, D), jnp.float32),
        jax.ShapeDtypeStruct((N, D), jnp.float32),
    ],
)


@jax.jit
def kernel(a_feature, b_feature, edge_index):
    ei = edge_index.astype(jnp.int32)
    src2 = ei[0].reshape(ROWS2, CH)
    dst2 = ei[1].reshape(ROWS2, CH)

    zeros_hist = jnp.zeros((N,), jnp.float32)
    deg_parts = (_deg_kernel(ei.reshape(2 * E), zeros_hist)
                 .reshape(NW, NB, NBLK).transpose(1, 0, 2))

    tab_a, tab_b, src_ofs, dst_ofs = _scale_call(
        deg_parts, a_feature, b_feature, src2, dst2)

    zeros_feat = jnp.zeros((APT, 32), jnp.float32)
    s_a, s_b = _spmm_kernel(
        tab_a.reshape(2 * N, 32), tab_b.reshape(2 * N, 32),
        src2, dst2, src_ofs, dst_ofs, zeros_feat)

    out_a, out_b = _combine_call(deg_parts, s_a, s_b, a_feature, b_feature)
    return out_a, out_b


# speedup vs baseline: 41.9395x; 1.0274x over previous
import functools

import jax
import jax.numpy as jnp
from jax import lax
from jax.experimental import pallas as pl
from jax.experimental.pallas import tpu as pltpu
from jax.experimental.pallas import tpu_sc as plsc

N = 50000
E = 800000
D = 64
EPS = 1e-08

NW = 32
TPS = 16
EPT1 = E // TPS
CH = 125
ROWS2 = E // CH
RPT = ROWS2 // TPS
APT = N // TPS
RCHUNK = 40
NBUF = 4
NB = 10
NBLK = N // NB
TROWS = 2 * N * 32 // 128
TBLK = NBLK * 32 // 128
SROWS = 2 * N * 32 // 128

_mesh = plsc.VectorSubcoreMesh(core_axis_name="c", subcore_axis_name="s")
_sc_params = pltpu.CompilerParams(needs_layout_passes=False,
                                  use_tc_tiling_on_sc=False)


@functools.partial(
    pl.kernel,
    out_type=(
        jax.ShapeDtypeStruct((NW * N,), jnp.float32),
        jax.ShapeDtypeStruct((2 * E,), jnp.int32),
        jax.ShapeDtypeStruct((2 * E,), jnp.int32),
    ),
    mesh=_mesh,
    compiler_params=_sc_params,
    scratch_types=[
        pltpu.VMEM((EPT1,), jnp.int32),
        pltpu.VMEM((N,), jnp.float32),
    ],
)
def _deg_kernel(edges_hbm, zeros_hbm, out_hbm, srcall_hbm, dstall_hbm,
                idx_v, hist_v):
    c = lax.axis_index("c")
    s = lax.axis_index("s")
    pltpu.sync_copy(edges_hbm.at[pl.ds(c * E + s * EPT1, EPT1)], idx_v)
    pltpu.sync_copy(zeros_hbm, hist_v)
    ones = jnp.full((16,), 1.0, dtype=jnp.float32)

    def body(i, carry):
        iv = idx_v[pl.ds(i * 16, 16)]
        plsc.addupdate_scatter(hist_v, [iv], ones)
        return carry

    lax.fori_loop(0, EPT1 // 16, body, 0)
    w = c * TPS + s
    for i in range(NB):
        pltpu.sync_copy(hist_v.at[pl.ds(i * NBLK, NBLK)],
                        out_hbm.at[pl.ds(i * NW * NBLK + w * NBLK, NBLK)])

    @pl.when(c == 0)
    def _():
        pltpu.sync_copy(idx_v, srcall_hbm.at[pl.ds(s * EPT1, EPT1)])

    @pl.when(c == 1)
    def _():
        pltpu.sync_copy(idx_v, dstall_hbm.at[pl.ds(s * EPT1, EPT1)])

    def addn(i, carry):
        idx_v[pl.ds(i * 16, 16)] = idx_v[pl.ds(i * 16, 16)] + N
        return carry

    lax.fori_loop(0, EPT1 // 16, addn, 0)

    @pl.when(c == 0)
    def _():
        pltpu.sync_copy(idx_v, srcall_hbm.at[pl.ds(E + s * EPT1, EPT1)])

    @pl.when(c == 1)
    def _():
        pltpu.sync_copy(idx_v, dstall_hbm.at[pl.ds(E + s * EPT1, EPT1)])


def _scale_body(dp_ref, a_ref, b_ref, ta_ref, tb_ref):
    h = pl.program_id(1)
    dp = dp_ref[0]
    deg_a = jnp.sum(dp[0:TPS, :], axis=0)
    deg_b = jnp.sum(dp[TPS:NW, :], axis=0)
    inv_a = 1.0 / (jnp.sqrt(deg_a) + EPS)
    inv_b = 1.0 / (jnp.sqrt(deg_b) + EPS)
    ta = a_ref[...] * inv_a[:, None]
    tb = b_ref[...] * inv_b[:, None]
    ta_h = jnp.where(h == 0, ta[:, 0:32], ta[:, 32:64])
    tb_h = jnp.where(h == 0, tb[:, 0:32], tb[:, 32:64])
    ta_ref[0] = ta_h
    tb_ref[0] = tb_h


_scale_call = pl.pallas_call(
    _scale_body,
    grid=(NB, 2),
    in_specs=[
        pl.BlockSpec((1, NW, NBLK), lambda i, h: (i, 0, 0)),
        pl.BlockSpec((NBLK, D), lambda i, h: (i, 0)),
        pl.BlockSpec((NBLK, D), lambda i, h: (i, 0)),
    ],
    out_specs=[
        pl.BlockSpec((1, NBLK, 32), lambda i, h: (h, i, 0)),
        pl.BlockSpec((1, NBLK, 32), lambda i, h: (h, i, 0)),
    ],
    out_shape=[
        jax.ShapeDtypeStruct((2, N, 32), jnp.float32),
        jax.ShapeDtypeStruct((2, N, 32), jnp.float32),
    ],
)


@functools.partial(
    pl.kernel,
    out_type=(
        jax.ShapeDtypeStruct((2, N, 32), jnp.float32),
        jax.ShapeDtypeStruct((2, N, 32), jnp.float32),
    ),
    mesh=_mesh,
    compiler_params=_sc_params,
    scratch_types=[
        pltpu.VMEM_SHARED((N, 32), jnp.float32),
        pltpu.VMEM((RCHUNK, CH), jnp.int32),
        pltpu.VMEM((RCHUNK, CH), jnp.int32),
        [pltpu.VMEM((CH, 32), jnp.float32) for _ in range(NBUF)],
        [pltpu.SemaphoreType.DMA for _ in range(NBUF)],
        [pltpu.SemaphoreType.DMA for _ in range(NBUF)],
    ],
)
def _spmm_kernel(tab_a, tab_b, src_all, dst_all, zeros_hbm,
                 sa_out, sb_out, acc, gidx, sidx, rows, gsem, ssem):
    c = lax.axis_index("c")
    s = lax.axis_index("s")
    row0 = s * RPT
    acc0 = s * APT

    def phase(tab_hbm, gall_hbm, sall_hbm, out_hbm):
        pltpu.sync_copy(zeros_hbm, acc.at[pl.ds(acc0, APT)])
        plsc.subcore_barrier()

        def outer(t, carry):
            r0 = row0 + t * RCHUNK
            pltpu.sync_copy(gall_hbm.at[c, pl.ds(r0, RCHUNK)], gidx)
            pltpu.sync_copy(sall_hbm.at[0, pl.ds(r0, RCHUNK)], sidx)

            def passes(q, carry2):
                j0 = q * NBUF
                for b in range(NBUF):
                    @pl.when(q > 0)
                    def _():
                        pltpu.make_async_copy(
                            rows[b], acc.at[sidx.at[0]], ssem[b]).wait()
                    pltpu.async_copy(
                        tab_hbm.at[gidx.at[j0 + b]], rows[b], gsem[b])
                for b in range(NBUF):
                    pltpu.make_async_copy(
                        tab_hbm.at[gidx.at[0]], rows[b], gsem[b]).wait()
                    pltpu.async_copy(
                        rows[b], acc.at[sidx.at[j0 + b]], ssem[b], add=True)
                return carry2

            lax.fori_loop(0, RCHUNK // NBUF, passes, carry)
            for b in range(NBUF):
                pltpu.make_async_copy(
                    rows[b], acc.at[sidx.at[0]], ssem[b]).wait()
            return carry

        lax.fori_loop(0, RPT // RCHUNK, outer, 0)
        plsc.subcore_barrier()
        pltpu.sync_copy(acc.at[pl.ds(acc0, APT)],
                        out_hbm.at[c, pl.ds(acc0, APT)])
        plsc.subcore_barrier()

    phase(tab_b, dst_all, src_all, sa_out)
    phase(tab_a, src_all, dst_all, sb_out)


def _combine_body(dp_ref, sa0_ref, sa1_ref, sb0_ref, sb1_ref,
                  a_ref, b_ref, oa_ref, ob_ref):
    dp = dp_ref[0]
    deg_a = jnp.sum(dp[0:TPS, :], axis=0)
    deg_b = jnp.sum(dp[TPS:NW, :], axis=0)
    inv_a = 1.0 / (jnp.sqrt(deg_a) + EPS)
    inv_b = 1.0 / (jnp.sqrt(deg_b) + EPS)
    sa = jnp.concatenate([sa0_ref[0], sa1_ref[0]], axis=1)
    sb = jnp.concatenate([sb0_ref[0], sb1_ref[0]], axis=1)
    oa_ref[...] = 0.5 * (a_ref[...] + inv_a[:, None] * sa)
    ob_ref[...] = 0.5 * (b_ref[...] + inv_b[:, None] * sb)


_combine_call = pl.pallas_call(
    _combine_body,
    grid=(NB,),
    in_specs=[
        pl.BlockSpec((1, NW, NBLK), lambda i: (i, 0, 0)),
        pl.BlockSpec((1, NBLK, 32), lambda i: (0, i, 0)),
        pl.BlockSpec((1, NBLK, 32), lambda i: (1, i, 0)),
        pl.BlockSpec((1, NBLK, 32), lambda i: (0, i, 0)),
        pl.BlockSpec((1, NBLK, 32), lambda i: (1, i, 0)),
        pl.BlockSpec((NBLK, D), lambda i: (i, 0)),
        pl.BlockSpec((NBLK, D), lambda i: (i, 0)),
    ],
    out_specs=[
        pl.BlockSpec((NBLK, D), lambda i: (i, 0)),
        pl.BlockSpec((NBLK, D), lambda i: (i, 0)),
    ],
    out_shape=[
        jax.ShapeDtypeStruct((N, D), jnp.float32),
        jax.ShapeDtypeStruct((N, D), jnp.float32),
    ],
)


@jax.jit
def kernel(a_feature, b_feature, edge_index):
    ei = edge_index.astype(jnp.int32)

    zeros_hist = jnp.zeros((N,), jnp.float32)
    deg_flat, src_all, dst_all = _deg_kernel(ei.reshape(2 * E), zeros_hist)
    deg_parts = deg_flat.reshape(NB, NW, NBLK)

    tab_a, tab_b = _scale_call(deg_parts, a_feature, b_feature)

    zeros_feat = jnp.zeros((APT, 32), jnp.float32)
    s_a, s_b = _spmm_kernel(
        tab_a.reshape(2 * N, 32), tab_b.reshape(2 * N, 32),
        src_all.reshape(2, ROWS2, CH), dst_all.reshape(2, ROWS2, CH),
        zeros_feat)

    out_a, out_b = _combine_call(
        deg_parts, s_a, s_a, s_b, s_b, a_feature, b_feature)
    return out_a, out_b


# speedup vs baseline: 42.9175x; 1.0233x over previous
import functools

import jax
import jax.numpy as jnp
from jax import lax
from jax.experimental import pallas as pl
from jax.experimental.pallas import tpu as pltpu
from jax.experimental.pallas import tpu_sc as plsc

N = 50000
E = 800000
D = 64
EPS = 1e-08

NW = 32
TPS = 16
EPT1 = E // TPS
CH = 125
ROWS2 = E // CH
RPT = ROWS2 // TPS
APT = N // TPS
RCHUNK = 40
NBUF = 5
NB = 10
NBLK = N // NB
TROWS = 2 * N * 32 // 128
TBLK = NBLK * 32 // 128
SROWS = 2 * N * 32 // 128

_mesh = plsc.VectorSubcoreMesh(core_axis_name="c", subcore_axis_name="s")
_sc_params = pltpu.CompilerParams(needs_layout_passes=False,
                                  use_tc_tiling_on_sc=False)


@functools.partial(
    pl.kernel,
    out_type=(
        jax.ShapeDtypeStruct((NW * N,), jnp.float32),
        jax.ShapeDtypeStruct((2 * E,), jnp.int32),
        jax.ShapeDtypeStruct((2 * E,), jnp.int32),
    ),
    mesh=_mesh,
    compiler_params=_sc_params,
    scratch_types=[
        pltpu.VMEM((EPT1,), jnp.int32),
        pltpu.VMEM((N,), jnp.float32),
    ],
)
def _deg_kernel(edges_hbm, zeros_hbm, out_hbm, srcall_hbm, dstall_hbm,
                idx_v, hist_v):
    c = lax.axis_index("c")
    s = lax.axis_index("s")
    pltpu.sync_copy(edges_hbm.at[pl.ds(c * E + s * EPT1, EPT1)], idx_v)
    pltpu.sync_copy(zeros_hbm, hist_v)
    ones = jnp.full((16,), 1.0, dtype=jnp.float32)

    def body(i, carry):
        iv = idx_v[pl.ds(i * 16, 16)]
        plsc.addupdate_scatter(hist_v, [iv], ones)
        return carry

    lax.fori_loop(0, EPT1 // 16, body, 0)
    w = c * TPS + s
    for i in range(NB):
        pltpu.sync_copy(hist_v.at[pl.ds(i * NBLK, NBLK)],
                        out_hbm.at[pl.ds(i * NW * NBLK + w * NBLK, NBLK)])

    @pl.when(c == 0)
    def _():
        pltpu.sync_copy(idx_v, srcall_hbm.at[pl.ds(s * EPT1, EPT1)])

    @pl.when(c == 1)
    def _():
        pltpu.sync_copy(idx_v, dstall_hbm.at[pl.ds(s * EPT1, EPT1)])

    def addn(i, carry):
        idx_v[pl.ds(i * 16, 16)] = idx_v[pl.ds(i * 16, 16)] + N
        return carry

    lax.fori_loop(0, EPT1 // 16, addn, 0)

    @pl.when(c == 0)
    def _():
        pltpu.sync_copy(idx_v, srcall_hbm.at[pl.ds(E + s * EPT1, EPT1)])

    @pl.when(c == 1)
    def _():
        pltpu.sync_copy(idx_v, dstall_hbm.at[pl.ds(E + s * EPT1, EPT1)])


def _scale_body(dp_ref, a_ref, b_ref, ta_ref, tb_ref):
    h = pl.program_id(1)
    dp = dp_ref[0]
    deg_a = jnp.sum(dp[0:TPS, :], axis=0)
    deg_b = jnp.sum(dp[TPS:NW, :], axis=0)
    inv_a = 1.0 / (jnp.sqrt(deg_a) + EPS)
    inv_b = 1.0 / (jnp.sqrt(deg_b) + EPS)
    ta = a_ref[...] * inv_a[:, None]
    tb = b_ref[...] * inv_b[:, None]
    ta_h = jnp.where(h == 0, ta[:, 0:32], ta[:, 32:64])
    tb_h = jnp.where(h == 0, tb[:, 0:32], tb[:, 32:64])
    ta_ref[0] = ta_h
    tb_ref[0] = tb_h


_scale_call = pl.pallas_call(
    _scale_body,
    grid=(NB, 2),
    in_specs=[
        pl.BlockSpec((1, NW, NBLK), lambda i, h: (i, 0, 0)),
        pl.BlockSpec((NBLK, D), lambda i, h: (i, 0)),
        pl.BlockSpec((NBLK, D), lambda i, h: (i, 0)),
    ],
    out_specs=[
        pl.BlockSpec((1, NBLK, 32), lambda i, h: (h, i, 0)),
        pl.BlockSpec((1, NBLK, 32), lambda i, h: (h, i, 0)),
    ],
    out_shape=[
        jax.ShapeDtypeStruct((2, N, 32), jnp.float32),
        jax.ShapeDtypeStruct((2, N, 32), jnp.float32),
    ],
)


@functools.partial(
    pl.kernel,
    out_type=(
        jax.ShapeDtypeStruct((2, N, 32), jnp.float32),
        jax.ShapeDtypeStruct((2, N, 32), jnp.float32),
    ),
    mesh=_mesh,
    compiler_params=_sc_params,
    scratch_types=[
        pltpu.VMEM_SHARED((N, 32), jnp.float32),
        pltpu.VMEM((RCHUNK, CH), jnp.int32),
        pltpu.VMEM((RCHUNK, CH), jnp.int32),
        [pltpu.VMEM((CH, 32), jnp.float32) for _ in range(NBUF)],
        [pltpu.SemaphoreType.DMA for _ in range(NBUF)],
        [pltpu.SemaphoreType.DMA for _ in range(NBUF)],
    ],
)
def _spmm_kernel(tab_a, tab_b, src_all, dst_all, zeros_hbm,
                 sa_out, sb_out, acc, gidx, sidx, rows, gsem, ssem):
    c = lax.axis_index("c")
    s = lax.axis_index("s")
    row0 = s * RPT
    acc0 = s * APT

    def phase(tab_hbm, gall_hbm, sall_hbm, out_hbm):
        pltpu.sync_copy(zeros_hbm, acc.at[pl.ds(acc0, APT)])
        plsc.subcore_barrier()

        def outer(t, carry):
            r0 = row0 + t * RCHUNK
            pltpu.sync_copy(gall_hbm.at[c, pl.ds(r0, RCHUNK)], gidx)
            pltpu.sync_copy(sall_hbm.at[0, pl.ds(r0, RCHUNK)], sidx)

            def passes(q, carry2):
                j0 = q * NBUF
                for b in range(NBUF):
                    @pl.when(q > 0)
                    def _():
                        pltpu.make_async_copy(
                            rows[b], acc.at[sidx.at[0]], ssem[b]).wait()
                    pltpu.async_copy(
                        tab_hbm.at[gidx.at[j0 + b]], rows[b], gsem[b])
                for b in range(NBUF):
                    pltpu.make_async_copy(
                        tab_hbm.at[gidx.at[0]], rows[b], gsem[b]).wait()
                    pltpu.async_copy(
                        rows[b], acc.at[sidx.at[j0 + b]], ssem[b], add=True)
                return carry2

            lax.fori_loop(0, RCHUNK // NBUF, passes, carry)
            for b in range(NBUF):
                pltpu.make_async_copy(
                    rows[b], acc.at[sidx.at[0]], ssem[b]).wait()
            return carry

        lax.fori_loop(0, RPT // RCHUNK, outer, 0)
        plsc.subcore_barrier()
        pltpu.sync_copy(acc.at[pl.ds(acc0, APT)],
                        out_hbm.at[c, pl.ds(acc0, APT)])
        plsc.subcore_barrier()

    phase(tab_b, dst_all, src_all, sa_out)
    phase(tab_a, src_all, dst_all, sb_out)


def _combine_body(dp_ref, sa0_ref, sa1_ref, sb0_ref, sb1_ref,
                  a_ref, b_ref, oa_ref, ob_ref):
    dp = dp_ref[0]
    deg_a = jnp.sum(dp[0:TPS, :], axis=0)
    deg_b = jnp.sum(dp[TPS:NW, :], axis=0)
    inv_a = 1.0 / (jnp.sqrt(deg_a) + EPS)
    inv_b = 1.0 / (jnp.sqrt(deg_b) + EPS)
    sa = jnp.concatenate([sa0_ref[0], sa1_ref[0]], axis=1)
    sb = jnp.concatenate([sb0_ref[0], sb1_ref[0]], axis=1)
    oa_ref[...] = 0.5 * (a_ref[...] + inv_a[:, None] * sa)
    ob_ref[...] = 0.5 * (b_ref[...] + inv_b[:, None] * sb)


_combine_call = pl.pallas_call(
    _combine_body,
    grid=(NB,),
    in_specs=[
        pl.BlockSpec((1, NW, NBLK), lambda i: (i, 0, 0)),
        pl.BlockSpec((1, NBLK, 32), lambda i: (0, i, 0)),
        pl.BlockSpec((1, NBLK, 32), lambda i: (1, i, 0)),
        pl.BlockSpec((1, NBLK, 32), lambda i: (0, i, 0)),
        pl.BlockSpec((1, NBLK, 32), lambda i: (1, i, 0)),
        pl.BlockSpec((NBLK, D), lambda i: (i, 0)),
        pl.BlockSpec((NBLK, D), lambda i: (i, 0)),
    ],
    out_specs=[
        pl.BlockSpec((NBLK, D), lambda i: (i, 0)),
        pl.BlockSpec((NBLK, D), lambda i: (i, 0)),
    ],
    out_shape=[
        jax.ShapeDtypeStruct((N, D), jnp.float32),
        jax.ShapeDtypeStruct((N, D), jnp.float32),
    ],
)


@jax.jit
def kernel(a_feature, b_feature, edge_index):
    ei = edge_index.astype(jnp.int32)

    zeros_hist = jnp.zeros((N,), jnp.float32)
    deg_flat, src_all, dst_all = _deg_kernel(ei.reshape(2 * E), zeros_hist)
    deg_parts = deg_flat.reshape(NB, NW, NBLK)

    tab_a, tab_b = _scale_call(deg_parts, a_feature, b_feature)

    zeros_feat = jnp.zeros((APT, 32), jnp.float32)
    s_a, s_b = _spmm_kernel(
        tab_a.reshape(2 * N, 32), tab_b.reshape(2 * N, 32),
        src_all.reshape(2, ROWS2, CH), dst_all.reshape(2, ROWS2, CH),
        zeros_feat)

    out_a, out_b = _combine_call(
        deg_parts, s_a, s_a, s_b, s_b, a_feature, b_feature)
    return out_a, out_b


# speedup vs baseline: 43.7131x; 1.0185x over previous
import functools

import jax
import jax.numpy as jnp
from jax import lax
from jax.experimental import pallas as pl
from jax.experimental.pallas import tpu as pltpu
from jax.experimental.pallas import tpu_sc as plsc

N = 50000
E = 800000
D = 64
EPS = 1e-08

NW = 32
TPS = 16
EPT1 = E // TPS
CH = 125
ROWS2 = E // CH
RPT = ROWS2 // TPS
APT = N // TPS
RCHUNK = 20
OUTER = RPT // RCHUNK
NBUF = 5
NB = 10
NBLK = N // NB
TROWS = 2 * N * 32 // 128
TBLK = NBLK * 32 // 128
SROWS = 2 * N * 32 // 128

_mesh = plsc.VectorSubcoreMesh(core_axis_name="c", subcore_axis_name="s")
_sc_params = pltpu.CompilerParams(needs_layout_passes=False,
                                  use_tc_tiling_on_sc=False)


@functools.partial(
    pl.kernel,
    out_type=(
        jax.ShapeDtypeStruct((NW * N,), jnp.float32),
        jax.ShapeDtypeStruct((2 * E,), jnp.int32),
        jax.ShapeDtypeStruct((2 * E,), jnp.int32),
    ),
    mesh=_mesh,
    compiler_params=_sc_params,
    scratch_types=[
        pltpu.VMEM((EPT1,), jnp.int32),
        pltpu.VMEM((N,), jnp.float32),
    ],
)
def _deg_kernel(edges_hbm, zeros_hbm, out_hbm, srcall_hbm, dstall_hbm,
                idx_v, hist_v):
    c = lax.axis_index("c")
    s = lax.axis_index("s")
    pltpu.sync_copy(edges_hbm.at[pl.ds(c * E + s * EPT1, EPT1)], idx_v)
    pltpu.sync_copy(zeros_hbm, hist_v)
    ones = jnp.full((16,), 1.0, dtype=jnp.float32)

    def body(i, carry):
        iv = idx_v[pl.ds(i * 16, 16)]
        plsc.addupdate_scatter(hist_v, [iv], ones)
        return carry

    lax.fori_loop(0, EPT1 // 16, body, 0)
    w = c * TPS + s
    for i in range(NB):
        pltpu.sync_copy(hist_v.at[pl.ds(i * NBLK, NBLK)],
                        out_hbm.at[pl.ds(i * NW * NBLK + w * NBLK, NBLK)])

    @pl.when(c == 0)
    def _():
        pltpu.sync_copy(idx_v, srcall_hbm.at[pl.ds(s * EPT1, EPT1)])

    @pl.when(c == 1)
    def _():
        pltpu.sync_copy(idx_v, dstall_hbm.at[pl.ds(s * EPT1, EPT1)])

    def addn(i, carry):
        idx_v[pl.ds(i * 16, 16)] = idx_v[pl.ds(i * 16, 16)] + N
        return carry

    lax.fori_loop(0, EPT1 // 16, addn, 0)

    @pl.when(c == 0)
    def _():
        pltpu.sync_copy(idx_v, srcall_hbm.at[pl.ds(E + s * EPT1, EPT1)])

    @pl.when(c == 1)
    def _():
        pltpu.sync_copy(idx_v, dstall_hbm.at[pl.ds(E + s * EPT1, EPT1)])


def _scale_body(dp_ref, a_ref, b_ref, ta_ref, tb_ref):
    h = pl.program_id(1)
    dp = dp_ref[0]
    deg_a = jnp.sum(dp[0:TPS, :], axis=0)
    deg_b = jnp.sum(dp[TPS:NW, :], axis=0)
    inv_a = 1.0 / (jnp.sqrt(deg_a) + EPS)
    inv_b = 1.0 / (jnp.sqrt(deg_b) + EPS)
    ta = a_ref[...] * inv_a[:, None]
    tb = b_ref[...] * inv_b[:, None]
    ta_h = jnp.where(h == 0, ta[:, 0:32], ta[:, 32:64])
    tb_h = jnp.where(h == 0, tb[:, 0:32], tb[:, 32:64])
    ta_ref[0] = ta_h
    tb_ref[0] = tb_h


_scale_call = pl.pallas_call(
    _scale_body,
    grid=(NB, 2),
    in_specs=[
        pl.BlockSpec((1, NW, NBLK), lambda i, h: (i, 0, 0)),
        pl.BlockSpec((NBLK, D), lambda i, h: (i, 0)),
        pl.BlockSpec((NBLK, D), lambda i, h: (i, 0)),
    ],
    out_specs=[
        pl.BlockSpec((1, NBLK, 32), lambda i, h: (h, i, 0)),
        pl.BlockSpec((1, NBLK, 32), lambda i, h: (h, i, 0)),
    ],
    out_shape=[
        jax.ShapeDtypeStruct((2, N, 32), jnp.float32),
        jax.ShapeDtypeStruct((2, N, 32), jnp.float32),
    ],
)


@functools.partial(
    pl.kernel,
    out_type=(
        jax.ShapeDtypeStruct((2, N, 32), jnp.float32),
        jax.ShapeDtypeStruct((2, N, 32), jnp.float32),
    ),
    mesh=_mesh,
    compiler_params=_sc_params,
    scratch_types=[
        pltpu.VMEM_SHARED((N, 32), jnp.float32),
        [pltpu.VMEM((RCHUNK, CH), jnp.int32) for _ in range(2)],
        [pltpu.VMEM((RCHUNK, CH), jnp.int32) for _ in range(2)],
        [pltpu.VMEM((CH, 32), jnp.float32) for _ in range(NBUF)],
        [pltpu.SemaphoreType.DMA for _ in range(NBUF)],
        [pltpu.SemaphoreType.DMA for _ in range(NBUF)],
        [pltpu.SemaphoreType.DMA for _ in range(2)],
    ],
)
def _spmm_kernel(tab_a, tab_b, src_all, dst_all, zeros_hbm,
                 sa_out, sb_out, acc, gidx2, sidx2, rows, gsem, ssem, lsem):
    c = lax.axis_index("c")
    s = lax.axis_index("s")
    row0 = s * RPT
    acc0 = s * APT

    def phase(tab_hbm, gall_hbm, sall_hbm, out_hbm):
        pltpu.sync_copy(zeros_hbm, acc.at[pl.ds(acc0, APT)])
        pltpu.async_copy(gall_hbm.at[c, pl.ds(row0, RCHUNK)], gidx2[0],
                         lsem[0])
        pltpu.async_copy(sall_hbm.at[0, pl.ds(row0, RCHUNK)], sidx2[0],
                         lsem[0])
        plsc.subcore_barrier()

        def outer2(u, carry):
            for p in range(2):
                t_cur = 2 * u + p
                gi = gidx2[p]
                si = sidx2[p]
                pltpu.make_async_copy(
                    gall_hbm.at[c, pl.ds(row0, RCHUNK)], gi, lsem[p]).wait()
                pltpu.make_async_copy(
                    sall_hbm.at[0, pl.ds(row0, RCHUNK)], si, lsem[p]).wait()

                @pl.when(t_cur + 1 < OUTER)
                def _():
                    rn = row0 + (t_cur + 1) * RCHUNK
                    pltpu.async_copy(gall_hbm.at[c, pl.ds(rn, RCHUNK)],
                                     gidx2[1 - p], lsem[1 - p])
                    pltpu.async_copy(sall_hbm.at[0, pl.ds(rn, RCHUNK)],
                                     sidx2[1 - p], lsem[1 - p])

                def passes(q, carry2):
                    j0 = q * NBUF
                    for b in range(NBUF):
                        @pl.when(q > 0)
                        def _():
                            pltpu.make_async_copy(
                                rows[b], acc.at[si.at[0]], ssem[b]).wait()
                        pltpu.async_copy(
                            tab_hbm.at[gi.at[j0 + b]], rows[b], gsem[b])
                    for b in range(NBUF):
                        pltpu.make_async_copy(
                            tab_hbm.at[gi.at[0]], rows[b], gsem[b]).wait()
                        pltpu.async_copy(rows[b], acc.at[si.at[j0 + b]],
                                         ssem[b], add=True)
                    return carry2

                lax.fori_loop(0, RCHUNK // NBUF, passes, carry)
                for b in range(NBUF):
                    pltpu.make_async_copy(
                        rows[b], acc.at[si.at[0]], ssem[b]).wait()
            return carry

        lax.fori_loop(0, OUTER // 2, outer2, 0)
        plsc.subcore_barrier()
        pltpu.sync_copy(acc.at[pl.ds(acc0, APT)],
                        out_hbm.at[c, pl.ds(acc0, APT)])
        plsc.subcore_barrier()

    phase(tab_b, dst_all, src_all, sa_out)
    phase(tab_a, src_all, dst_all, sb_out)


def _combine_body(dp_ref, sa0_ref, sa1_ref, sb0_ref, sb1_ref,
                  a_ref, b_ref, oa_ref, ob_ref):
    dp = dp_ref[0]
    deg_a = jnp.sum(dp[0:TPS, :], axis=0)
    deg_b = jnp.sum(dp[TPS:NW, :], axis=0)
    inv_a = 1.0 / (jnp.sqrt(deg_a) + EPS)
    inv_b = 1.0 / (jnp.sqrt(deg_b) + EPS)
    sa = jnp.concatenate([sa0_ref[0], sa1_ref[0]], axis=1)
    sb = jnp.concatenate([sb0_ref[0], sb1_ref[0]], axis=1)
    oa_ref[...] = 0.5 * (a_ref[...] + inv_a[:, None] * sa)
    ob_ref[...] = 0.5 * (b_ref[...] + inv_b[:, None] * sb)


_combine_call = pl.pallas_call(
    _combine_body,
    grid=(NB,),
    in_specs=[
        pl.BlockSpec((1, NW, NBLK), lambda i: (i, 0, 0)),
        pl.BlockSpec((1, NBLK, 32), lambda i: (0, i, 0)),
        pl.BlockSpec((1, NBLK, 32), lambda i: (1, i, 0)),
        pl.BlockSpec((1, NBLK, 32), lambda i: (0, i, 0)),
        pl.BlockSpec((1, NBLK, 32), lambda i: (1, i, 0)),
        pl.BlockSpec((NBLK, D), lambda i: (i, 0)),
        pl.BlockSpec((NBLK, D), lambda i: (i, 0)),
    ],
    out_specs=[
        pl.BlockSpec((NBLK, D), lambda i: (i, 0)),
        pl.BlockSpec((NBLK, D), lambda i: (i, 0)),
    ],
    out_shape=[
        jax.ShapeDtypeStruct((N, D), jnp.float32),
        jax.ShapeDtypeStruct((N, D), jnp.float32),
    ],
)


@jax.jit
def kernel(a_feature, b_feature, edge_index):
    ei = edge_index.astype(jnp.int32)

    zeros_hist = jnp.zeros((N,), jnp.float32)
    deg_flat, src_all, dst_all = _deg_kernel(ei.reshape(2 * E), zeros_hist)
    deg_parts = deg_flat.reshape(NB, NW, NBLK)

    tab_a, tab_b = _scale_call(deg_parts, a_feature, b_feature)

    zeros_feat = jnp.zeros((APT, 32), jnp.float32)
    s_a, s_b = _spmm_kernel(
        tab_a.reshape(2 * N, 32), tab_b.reshape(2 * N, 32),
        src_all.reshape(2, ROWS2, CH), dst_all.reshape(2, ROWS2, CH),
        zeros_feat)

    out_a, out_b = _combine_call(
        deg_parts, s_a, s_a, s_b, s_b, a_feature, b_feature)
    return out_a, out_b


# speedup vs baseline: 45.3113x; 1.0366x over previous
import functools

import jax
import jax.numpy as jnp
from jax import lax
from jax.experimental import pallas as pl
from jax.experimental.pallas import tpu as pltpu
from jax.experimental.pallas import tpu_sc as plsc

N = 50000
E = 800000
D = 64
EPS = 1e-08

NW = 32
TPS = 16
EPT1 = E // TPS
CH = 125
ROWS2 = E // CH
RPT = ROWS2 // TPS
APT = N // TPS
RCHUNK = 20
OUTER = RPT // RCHUNK
NBUF = 5
NB = 10
NBLK = N // NB
TROWS = 2 * N * 32 // 128
TBLK = NBLK * 32 // 128
SROWS = 2 * N * 32 // 128

_mesh = plsc.VectorSubcoreMesh(core_axis_name="c", subcore_axis_name="s")
_sc_params = pltpu.CompilerParams(needs_layout_passes=False,
                                  use_tc_tiling_on_sc=False)


@functools.partial(
    pl.kernel,
    out_type=(
        jax.ShapeDtypeStruct((NW * N,), jnp.float32),
        jax.ShapeDtypeStruct((2 * E,), jnp.int32),
        jax.ShapeDtypeStruct((2 * E,), jnp.int32),
    ),
    mesh=_mesh,
    compiler_params=_sc_params,
    scratch_types=[
        pltpu.VMEM((EPT1,), jnp.int32),
        pltpu.VMEM((N,), jnp.float32),
    ],
)
def _deg_kernel(edges_hbm, zeros_hbm, out_hbm, srcall_hbm, dstall_hbm,
                idx_v, hist_v):
    c = lax.axis_index("c")
    s = lax.axis_index("s")
    pltpu.sync_copy(edges_hbm.at[pl.ds(c * E + s * EPT1, EPT1)], idx_v)
    pltpu.sync_copy(zeros_hbm, hist_v)
    ones = jnp.full((16,), 1.0, dtype=jnp.float32)

    def body(i, carry):
        iv = idx_v[pl.ds(i * 16, 16)]
        plsc.addupdate_scatter(hist_v, [iv], ones)
        return carry

    lax.fori_loop(0, EPT1 // 16, body, 0)
    w = c * TPS + s
    for i in range(NB):
        pltpu.sync_copy(hist_v.at[pl.ds(i * NBLK, NBLK)],
                        out_hbm.at[pl.ds(i * NW * NBLK + w * NBLK, NBLK)])

    @pl.when(c == 0)
    def _():
        pltpu.sync_copy(idx_v, srcall_hbm.at[pl.ds(s * EPT1, EPT1)])

    @pl.when(c == 1)
    def _():
        pltpu.sync_copy(idx_v, dstall_hbm.at[pl.ds(s * EPT1, EPT1)])

    def addn(i, carry):
        idx_v[pl.ds(i * 16, 16)] = idx_v[pl.ds(i * 16, 16)] + N
        return carry

    lax.fori_loop(0, EPT1 // 16, addn, 0)

    @pl.when(c == 0)
    def _():
        pltpu.sync_copy(idx_v, srcall_hbm.at[pl.ds(E + s * EPT1, EPT1)])

    @pl.when(c == 1)
    def _():
        pltpu.sync_copy(idx_v, dstall_hbm.at[pl.ds(E + s * EPT1, EPT1)])


def _scale_body(dp_ref, a_ref, b_ref, ta_ref, tb_ref):
    dp = dp_ref[0]
    deg_a = jnp.sum(dp[0:TPS, :], axis=0)
    deg_b = jnp.sum(dp[TPS:NW, :], axis=0)
    inv_a = 1.0 / (jnp.sqrt(deg_a) + EPS)
    inv_b = 1.0 / (jnp.sqrt(deg_b) + EPS)
    ta = a_ref[...] * inv_a[:, None]
    tb = b_ref[...] * inv_b[:, None]
    ta_ref[0] = ta[:, 0:32]
    ta_ref[1] = ta[:, 32:64]
    tb_ref[0] = tb[:, 0:32]
    tb_ref[1] = tb[:, 32:64]


_scale_call = pl.pallas_call(
    _scale_body,
    grid=(NB,),
    in_specs=[
        pl.BlockSpec((1, NW, NBLK), lambda i: (i, 0, 0)),
        pl.BlockSpec((NBLK, D), lambda i: (i, 0)),
        pl.BlockSpec((NBLK, D), lambda i: (i, 0)),
    ],
    out_specs=[
        pl.BlockSpec((2, NBLK, 32), lambda i: (0, i, 0)),
        pl.BlockSpec((2, NBLK, 32), lambda i: (0, i, 0)),
    ],
    out_shape=[
        jax.ShapeDtypeStruct((2, N, 32), jnp.float32),
        jax.ShapeDtypeStruct((2, N, 32), jnp.float32),
    ],
)


@functools.partial(
    pl.kernel,
    out_type=(
        jax.ShapeDtypeStruct((2, N, 32), jnp.float32),
        jax.ShapeDtypeStruct((2, N, 32), jnp.float32),
    ),
    mesh=_mesh,
    compiler_params=_sc_params,
    scratch_types=[
        pltpu.VMEM_SHARED((N, 32), jnp.float32),
        [pltpu.VMEM((RCHUNK, CH), jnp.int32) for _ in range(2)],
        [pltpu.VMEM((RCHUNK, CH), jnp.int32) for _ in range(2)],
        [pltpu.VMEM((CH, 32), jnp.float32) for _ in range(NBUF)],
        [pltpu.SemaphoreType.DMA for _ in range(NBUF)],
        [pltpu.SemaphoreType.DMA for _ in range(NBUF)],
        [pltpu.SemaphoreType.DMA for _ in range(2)],
    ],
)
def _spmm_kernel(tab_a, tab_b, src_all, dst_all, zeros_hbm,
                 sa_out, sb_out, acc, gidx2, sidx2, rows, gsem, ssem, lsem):
    c = lax.axis_index("c")
    s = lax.axis_index("s")
    row0 = s * RPT
    acc0 = s * APT

    def phase(tab_hbm, gall_hbm, sall_hbm, out_hbm):
        pltpu.sync_copy(zeros_hbm, acc.at[pl.ds(acc0, APT)])
        pltpu.async_copy(gall_hbm.at[c, pl.ds(row0, RCHUNK)], gidx2[0],
                         lsem[0])
        pltpu.async_copy(sall_hbm.at[0, pl.ds(row0, RCHUNK)], sidx2[0],
                         lsem[0])
        plsc.subcore_barrier()

        def outer2(u, carry):
            for p in range(2):
                t_cur = 2 * u + p
                gi = gidx2[p]
                si = sidx2[p]
                pltpu.make_async_copy(
                    gall_hbm.at[c, pl.ds(row0, RCHUNK)], gi, lsem[p]).wait()
                pltpu.make_async_copy(
                    sall_hbm.at[0, pl.ds(row0, RCHUNK)], si, lsem[p]).wait()

                @pl.when(t_cur + 1 < OUTER)
                def _():
                    rn = row0 + (t_cur + 1) * RCHUNK
                    pltpu.async_copy(gall_hbm.at[c, pl.ds(rn, RCHUNK)],
                                     gidx2[1 - p], lsem[1 - p])
                    pltpu.async_copy(sall_hbm.at[0, pl.ds(rn, RCHUNK)],
                                     sidx2[1 - p], lsem[1 - p])

                def passes(q, carry2):
                    j0 = q * NBUF
                    for b in range(NBUF):
                        @pl.when(q > 0)
                        def _():
                            pltpu.make_async_copy(
                                rows[b], acc.at[si.at[0]], ssem[b]).wait()
                        pltpu.async_copy(
                            tab_hbm.at[gi.at[j0 + b]], rows[b], gsem[b])
                    for b in range(NBUF):
                        pltpu.make_async_copy(
                            tab_hbm.at[gi.at[0]], rows[b], gsem[b]).wait()
                        pltpu.async_copy(rows[b], acc.at[si.at[j0 + b]],
                                         ssem[b], add=True)
                    return carry2

                lax.fori_loop(0, RCHUNK // NBUF, passes, carry)
                for b in range(NBUF):
                    pltpu.make_async_copy(
                        rows[b], acc.at[si.at[0]], ssem[b]).wait()
            return carry

        lax.fori_loop(0, OUTER // 2, outer2, 0)
        plsc.subcore_barrier()
        pltpu.sync_copy(acc.at[pl.ds(acc0, APT)],
                        out_hbm.at[c, pl.ds(acc0, APT)])
        plsc.subcore_barrier()

    phase(tab_b, dst_all, src_all, sa_out)
    phase(tab_a, src_all, dst_all, sb_out)


def _combine_body(dp_ref, sa0_ref, sa1_ref, sb0_ref, sb1_ref,
                  a_ref, b_ref, oa_ref, ob_ref):
    dp = dp_ref[0]
    deg_a = jnp.sum(dp[0:TPS, :], axis=0)
    deg_b = jnp.sum(dp[TPS:NW, :], axis=0)
    inv_a = 1.0 / (jnp.sqrt(deg_a) + EPS)
    inv_b = 1.0 / (jnp.sqrt(deg_b) + EPS)
    sa = jnp.concatenate([sa0_ref[0], sa1_ref[0]], axis=1)
    sb = jnp.concatenate([sb0_ref[0], sb1_ref[0]], axis=1)
    oa_ref[...] = 0.5 * (a_ref[...] + inv_a[:, None] * sa)
    ob_ref[...] = 0.5 * (b_ref[...] + inv_b[:, None] * sb)


_combine_call = pl.pallas_call(
    _combine_body,
    grid=(NB,),
    in_specs=[
        pl.BlockSpec((1, NW, NBLK), lambda i: (i, 0, 0)),
        pl.BlockSpec((1, NBLK, 32), lambda i: (0, i, 0)),
        pl.BlockSpec((1, NBLK, 32), lambda i: (1, i, 0)),
        pl.BlockSpec((1, NBLK, 32), lambda i: (0, i, 0)),
        pl.BlockSpec((1, NBLK, 32), lambda i: (1, i, 0)),
        pl.BlockSpec((NBLK, D), lambda i: (i, 0)),
        pl.BlockSpec((NBLK, D), lambda i: (i, 0)),
    ],
    out_specs=[
        pl.BlockSpec((NBLK, D), lambda i: (i, 0)),
        pl.BlockSpec((NBLK, D), lambda i: (i, 0)),
    ],
    out_shape=[
        jax.ShapeDtypeStruct((N, D), jnp.float32),
        jax.ShapeDtypeStruct((N, D), jnp.float32),
    ],
)


@jax.jit
def kernel(a_feature, b_feature, edge_index):
    ei = edge_index.astype(jnp.int32)

    zeros_hist = jnp.zeros((N,), jnp.float32)
    deg_flat, src_all, dst_all = _deg_kernel(ei.reshape(2 * E), zeros_hist)
    deg_parts = deg_flat.reshape(NB, NW, NBLK)

    tab_a, tab_b = _scale_call(deg_parts, a_feature, b_feature)

    zeros_feat = jnp.zeros((APT, 32), jnp.float32)
    s_a, s_b = _spmm_kernel(
        tab_a.reshape(2 * N, 32), tab_b.reshape(2 * N, 32),
        src_all.reshape(2, ROWS2, CH), dst_all.reshape(2, ROWS2, CH),
        zeros_feat)

    out_a, out_b = _combine_call(
        deg_parts, s_a, s_a, s_b, s_b, a_feature, b_feature)
    return out_a, out_b


# speedup vs baseline: 45.3710x; 1.0013x over previous
import functools

import jax
import jax.numpy as jnp
from jax import lax
from jax.experimental import pallas as pl
from jax.experimental.pallas import tpu as pltpu
from jax.experimental.pallas import tpu_sc as plsc

N = 50000
E = 800000
D = 64
EPS = 1e-08

NW = 32
TPS = 16
EPT1 = E // TPS
CH = 125
ROWS2 = E // CH
RPT = ROWS2 // TPS
APT = N // TPS
RCHUNK = 20
OUTER = RPT // RCHUNK
NBUF = 5
NB = 10
NBLK = N // NB

_mesh = plsc.VectorSubcoreMesh(core_axis_name="c", subcore_axis_name="s")
_sc_params = pltpu.CompilerParams(needs_layout_passes=False,
                                  use_tc_tiling_on_sc=False)


@functools.partial(
    pl.kernel,
    out_type=(
        jax.ShapeDtypeStruct((NW * N,), jnp.float32),
        jax.ShapeDtypeStruct((2 * E,), jnp.int32),
        jax.ShapeDtypeStruct((2 * E,), jnp.int32),
    ),
    mesh=_mesh,
    compiler_params=_sc_params,
    scratch_types=[
        pltpu.VMEM((EPT1,), jnp.int32),
        pltpu.VMEM((N,), jnp.float32),
    ],
)
def _deg_kernel(edges_hbm, zeros_hbm, out_hbm, srcall_hbm, dstall_hbm,
                idx_v, hist_v):
    c = lax.axis_index("c")
    s = lax.axis_index("s")
    pltpu.sync_copy(edges_hbm.at[pl.ds(c * E + s * EPT1, EPT1)], idx_v)
    pltpu.sync_copy(zeros_hbm, hist_v)
    ones = jnp.full((16,), 1.0, dtype=jnp.float32)

    def body(i, carry):
        iv = idx_v[pl.ds(i * 16, 16)]
        plsc.addupdate_scatter(hist_v, [iv], ones)
        return carry

    lax.fori_loop(0, EPT1 // 16, body, 0)
    w = c * TPS + s
    for i in range(NB):
        pltpu.sync_copy(hist_v.at[pl.ds(i * NBLK, NBLK)],
                        out_hbm.at[pl.ds(i * NW * NBLK + w * NBLK, NBLK)])

    @pl.when(c == 0)
    def _():
        pltpu.sync_copy(idx_v, srcall_hbm.at[pl.ds(s * EPT1, EPT1)])

    @pl.when(c == 1)
    def _():
        pltpu.sync_copy(idx_v, dstall_hbm.at[pl.ds(s * EPT1, EPT1)])

    def addn(i, carry):
        idx_v[pl.ds(i * 16, 16)] = idx_v[pl.ds(i * 16, 16)] + N
        return carry

    lax.fori_loop(0, EPT1 // 16, addn, 0)

    @pl.when(c == 0)
    def _():
        pltpu.sync_copy(idx_v, srcall_hbm.at[pl.ds(E + s * EPT1, EPT1)])

    @pl.when(c == 1)
    def _():
        pltpu.sync_copy(idx_v, dstall_hbm.at[pl.ds(E + s * EPT1, EPT1)])


def _scale_body(dp_ref, a_ref, b_ref, ta_ref, tb_ref):
    dp = dp_ref[0]
    deg_a = jnp.sum(dp[0:TPS, :], axis=0)
    deg_b = jnp.sum(dp[TPS:NW, :], axis=0)
    inv_a = 1.0 / (jnp.sqrt(deg_a) + EPS)
    inv_b = 1.0 / (jnp.sqrt(deg_b) + EPS)
    ta = a_ref[...] * inv_a[:, None]
    tb = b_ref[...] * inv_b[:, None]
    ta_ref[0] = ta[:, 0:32]
    ta_ref[1] = ta[:, 32:64]
    tb_ref[0] = tb[:, 0:32]
    tb_ref[1] = tb[:, 32:64]


_scale_call = pl.pallas_call(
    _scale_body,
    grid=(NB,),
    in_specs=[
        pl.BlockSpec((1, NW, NBLK), lambda i: (i, 0, 0)),
        pl.BlockSpec((NBLK, D), lambda i: (i, 0)),
        pl.BlockSpec((NBLK, D), lambda i: (i, 0)),
    ],
    out_specs=[
        pl.BlockSpec((2, NBLK, 32), lambda i: (0, i, 0)),
        pl.BlockSpec((2, NBLK, 32), lambda i: (0, i, 0)),
    ],
    out_shape=[
        jax.ShapeDtypeStruct((2, N, 32), jnp.float32),
        jax.ShapeDtypeStruct((2, N, 32), jnp.float32),
    ],
)


@functools.partial(
    pl.kernel,
    out_type=(
        jax.ShapeDtypeStruct((2, N, 32), jnp.float32),
        jax.ShapeDtypeStruct((2, N, 32), jnp.float32),
    ),
    mesh=_mesh,
    compiler_params=_sc_params,
    scratch_types=[
        pltpu.VMEM_SHARED((N, 32), jnp.float32),
        [pltpu.VMEM((RCHUNK, CH), jnp.int32) for _ in range(2)],
        [pltpu.VMEM((RCHUNK, CH), jnp.int32) for _ in range(2)],
        [pltpu.VMEM((CH, 32), jnp.float32) for _ in range(NBUF)],
        [pltpu.SemaphoreType.DMA for _ in range(NBUF)],
        [pltpu.SemaphoreType.DMA for _ in range(NBUF)],
        [pltpu.SemaphoreType.DMA for _ in range(2)],
    ],
)
def _spmm_kernel(tab_a, tab_b, src_all, dst_all, zeros_hbm,
                 sa_out, sb_out, acc, gidx2, sidx2, rows, gsem, ssem, lsem):
    c = lax.axis_index("c")
    s = lax.axis_index("s")
    row0 = s * RPT
    acc0 = s * APT

    def phase(tab_hbm, gall_hbm, sall_hbm, out_hbm):
        pltpu.sync_copy(zeros_hbm, acc.at[pl.ds(acc0, APT)])
        pltpu.async_copy(gall_hbm.at[c, pl.ds(row0, RCHUNK)], gidx2[0],
                         lsem[0])
        pltpu.async_copy(sall_hbm.at[0, pl.ds(row0, RCHUNK)], sidx2[0],
                         lsem[0])
        plsc.subcore_barrier()

        def outer2(u, carry):
            for p in range(2):
                t_cur = 2 * u + p
                gi = gidx2[p]
                si = sidx2[p]
                pltpu.make_async_copy(
                    gall_hbm.at[c, pl.ds(row0, RCHUNK)], gi, lsem[p]).wait()
                pltpu.make_async_copy(
                    sall_hbm.at[0, pl.ds(row0, RCHUNK)], si, lsem[p]).wait()

                @pl.when(t_cur + 1 < OUTER)
                def _():
                    rn = row0 + (t_cur + 1) * RCHUNK
                    pltpu.async_copy(gall_hbm.at[c, pl.ds(rn, RCHUNK)],
                                     gidx2[1 - p], lsem[1 - p])
                    pltpu.async_copy(sall_hbm.at[0, pl.ds(rn, RCHUNK)],
                                     sidx2[1 - p], lsem[1 - p])

                def passes(q, carry2):
                    j0 = q * NBUF
                    for b in range(NBUF):
                        @pl.when(q > 0)
                        def _():
                            pltpu.make_async_copy(
                                rows[b], acc.at[si.at[0]], ssem[b]).wait()
                        pltpu.async_copy(
                            tab_hbm.at[gi.at[j0 + b]], rows[b], gsem[b])
                    for b in range(NBUF):
                        pltpu.make_async_copy(
                            tab_hbm.at[gi.at[0]], rows[b], gsem[b]).wait()
                        pltpu.async_copy(rows[b], acc.at[si.at[j0 + b]],
                                         ssem[b], add=True)
                    return carry2

                lax.fori_loop(0, RCHUNK // NBUF, passes, carry)
                for b in range(NBUF):
                    pltpu.make_async_copy(
                        rows[b], acc.at[si.at[0]], ssem[b]).wait()
            return carry

        lax.fori_loop(0, OUTER // 2, outer2, 0)
        plsc.subcore_barrier()
        pltpu.sync_copy(acc.at[pl.ds(acc0, APT)],
                        out_hbm.at[c, pl.ds(acc0, APT)])
        plsc.subcore_barrier()

    phase(tab_b, dst_all, src_all, sa_out)
    phase(tab_a, src_all, dst_all, sb_out)


def _combine_body(dp_ref, sa0_ref, sa1_ref, sb0_ref, sb1_ref,
                  a_ref, b_ref, oa_ref, ob_ref):
    dp = dp_ref[0]
    deg_a = jnp.sum(dp[0:TPS, :], axis=0)
    deg_b = jnp.sum(dp[TPS:NW, :], axis=0)
    inv_a = 1.0 / (jnp.sqrt(deg_a) + EPS)
    inv_b = 1.0 / (jnp.sqrt(deg_b) + EPS)
    sa = jnp.concatenate([sa0_ref[0], sa1_ref[0]], axis=1)
    sb = jnp.concatenate([sb0_ref[0], sb1_ref[0]], axis=1)
    oa_ref[...] = 0.5 * (a_ref[...] + inv_a[:, None] * sa)
    ob_ref[...] = 0.5 * (b_ref[...] + inv_b[:, None] * sb)


_combine_call = pl.pallas_call(
    _combine_body,
    grid=(NB,),
    in_specs=[
        pl.BlockSpec((1, NW, NBLK), lambda i: (i, 0, 0)),
        pl.BlockSpec((1, NBLK, 32), lambda i: (0, i, 0)),
        pl.BlockSpec((1, NBLK, 32), lambda i: (1, i, 0)),
        pl.BlockSpec((1, NBLK, 32), lambda i: (0, i, 0)),
        pl.BlockSpec((1, NBLK, 32), lambda i: (1, i, 0)),
        pl.BlockSpec((NBLK, D), lambda i: (i, 0)),
        pl.BlockSpec((NBLK, D), lambda i: (i, 0)),
    ],
    out_specs=[
        pl.BlockSpec((NBLK, D), lambda i: (i, 0)),
        pl.BlockSpec((NBLK, D), lambda i: (i, 0)),
    ],
    out_shape=[
        jax.ShapeDtypeStruct((N, D), jnp.float32),
        jax.ShapeDtypeStruct((N, D), jnp.float32),
    ],
)


@jax.jit
def kernel(a_feature, b_feature, edge_index):
    ei = edge_index.astype(jnp.int32)

    zeros_hist = jnp.zeros((N,), jnp.float32)
    deg_flat, src_all, dst_all = _deg_kernel(ei.reshape(2 * E), zeros_hist)
    deg_parts = deg_flat.reshape(NB, NW, NBLK)

    tab_a, tab_b = _scale_call(deg_parts, a_feature, b_feature)

    zeros_feat = jnp.zeros((APT, 32), jnp.float32)
    s_a, s_b = _spmm_kernel(
        tab_a.reshape(2 * N, 32), tab_b.reshape(2 * N, 32),
        src_all.reshape(2, ROWS2, CH), dst_all.reshape(2, ROWS2, CH),
        zeros_feat)

    out_a, out_b = _combine_call(
        deg_parts, s_a, s_a, s_b, s_b, a_feature, b_feature)
    return out_a, out_b


# speedup vs baseline: 50.9393x; 1.1227x over previous
import functools

import jax
import jax.numpy as jnp
from jax import lax
from jax.experimental import pallas as pl
from jax.experimental.pallas import tpu as pltpu
from jax.experimental.pallas import tpu_sc as plsc

N = 50000
E = 800000
D = 64
EPS = 1e-08

NW = 32
TPS = 16
EPT1 = E // TPS
CH = 125
ROWS2 = E // CH
RPT = ROWS2 // TPS
APT = N // TPS
RCHUNK = 20
OUTER = RPT // RCHUNK
NBUF = 5
NB = 10
NBLK = N // NB

_mesh = plsc.VectorSubcoreMesh(core_axis_name="c", subcore_axis_name="s")
_sc_params = pltpu.CompilerParams(needs_layout_passes=False,
                                  use_tc_tiling_on_sc=False)


@functools.partial(
    pl.kernel,
    out_type=(
        jax.ShapeDtypeStruct((NW * N,), jnp.float32),
        jax.ShapeDtypeStruct((2 * E,), jnp.int32),
        jax.ShapeDtypeStruct((2 * E,), jnp.int32),
    ),
    mesh=_mesh,
    compiler_params=_sc_params,
    scratch_types=[
        pltpu.VMEM((EPT1,), jnp.int32),
        pltpu.VMEM((N,), jnp.float32),
    ],
)
def _deg_kernel(edges_hbm, zeros_hbm, out_hbm, srcall_hbm, dstall_hbm,
                idx_v, hist_v):
    c = lax.axis_index("c")
    s = lax.axis_index("s")
    pltpu.sync_copy(edges_hbm.at[pl.ds(c * E + s * EPT1, EPT1)], idx_v)
    pltpu.sync_copy(zeros_hbm, hist_v)
    ones = jnp.full((16,), 1.0, dtype=jnp.float32)

    def body(i, carry):
        iv = idx_v[pl.ds(i * 16, 16)]
        plsc.addupdate_scatter(hist_v, [iv], ones)
        return carry

    lax.fori_loop(0, EPT1 // 16, body, 0)
    w = c * TPS + s
    for i in range(NB):
        pltpu.sync_copy(hist_v.at[pl.ds(i * NBLK, NBLK)],
                        out_hbm.at[pl.ds(i * NW * NBLK + w * NBLK, NBLK)])

    @pl.when(c == 0)
    def _():
        pltpu.sync_copy(idx_v, srcall_hbm.at[pl.ds(s * EPT1, EPT1)])

    @pl.when(c == 1)
    def _():
        pltpu.sync_copy(idx_v, dstall_hbm.at[pl.ds(s * EPT1, EPT1)])

    def addn(i, carry):
        idx_v[pl.ds(i * 16, 16)] = idx_v[pl.ds(i * 16, 16)] + N
        return carry

    lax.fori_loop(0, EPT1 // 16, addn, 0)

    @pl.when(c == 0)
    def _():
        pltpu.sync_copy(idx_v, srcall_hbm.at[pl.ds(E + s * EPT1, EPT1)])

    @pl.when(c == 1)
    def _():
        pltpu.sync_copy(idx_v, dstall_hbm.at[pl.ds(E + s * EPT1, EPT1)])


def _scale_body(dp_ref, a_ref, b_ref, ta_ref, tb_ref):
    dp = dp_ref[0]
    deg_a = jnp.sum(dp[0:TPS, :], axis=0)
    deg_b = jnp.sum(dp[TPS:NW, :], axis=0)
    inv_a = 1.0 / (jnp.sqrt(deg_a) + EPS)
    inv_b = 1.0 / (jnp.sqrt(deg_b) + EPS)
    ta = a_ref[...] * inv_a[:, None]
    tb = b_ref[...] * inv_b[:, None]
    ta_ref[0] = ta[:, 0:32]
    ta_ref[1] = ta[:, 32:64]
    tb_ref[0] = tb[:, 0:32]
    tb_ref[1] = tb[:, 32:64]


_scale_call = pl.pallas_call(
    _scale_body,
    grid=(NB,),
    in_specs=[
        pl.BlockSpec((1, NW, NBLK), lambda i: (i, 0, 0)),
        pl.BlockSpec((NBLK, D), lambda i: (i, 0)),
        pl.BlockSpec((NBLK, D), lambda i: (i, 0)),
    ],
    out_specs=[
        pl.BlockSpec((2, NBLK, 32), lambda i: (0, i, 0)),
        pl.BlockSpec((2, NBLK, 32), lambda i: (0, i, 0)),
    ],
    out_shape=[
        jax.ShapeDtypeStruct((2, N, 32), jnp.float32),
        jax.ShapeDtypeStruct((2, N, 32), jnp.float32),
    ],
)


@functools.partial(
    pl.kernel,
    out_type=jax.ShapeDtypeStruct((2, N, 32), jnp.float32),
    mesh=_mesh,
    compiler_params=_sc_params,
    scratch_types=[
        pltpu.VMEM_SHARED((N, 32), jnp.float32),
        [pltpu.VMEM((RCHUNK, CH), jnp.int32) for _ in range(2)],
        [pltpu.VMEM((RCHUNK, CH), jnp.int32) for _ in range(2)],
        [pltpu.VMEM((CH, 32), jnp.float32) for _ in range(NBUF)],
        [pltpu.SemaphoreType.DMA for _ in range(NBUF)],
        [pltpu.SemaphoreType.DMA for _ in range(NBUF)],
        [pltpu.SemaphoreType.DMA for _ in range(2)],
    ],
)
def _spmm_phase(tab_hbm, gall_hbm, sall_hbm, zeros_hbm,
                out_hbm, acc, gidx2, sidx2, rows, gsem, ssem, lsem):
    c = lax.axis_index("c")
    s = lax.axis_index("s")
    row0 = s * RPT
    acc0 = s * APT

    if True:
        pltpu.sync_copy(zeros_hbm, acc.at[pl.ds(acc0, APT)])
        pltpu.async_copy(gall_hbm.at[c, pl.ds(row0, RCHUNK)], gidx2[0],
                         lsem[0])
        pltpu.async_copy(sall_hbm.at[0, pl.ds(row0, RCHUNK)], sidx2[0],
                         lsem[0])
        plsc.subcore_barrier()

        def outer2(u, carry):
            for p in range(2):
                t_cur = 2 * u + p
                gi = gidx2[p]
                si = sidx2[p]
                pltpu.make_async_copy(
                    gall_hbm.at[c, pl.ds(row0, RCHUNK)], gi, lsem[p]).wait()
                pltpu.make_async_copy(
                    sall_hbm.at[0, pl.ds(row0, RCHUNK)], si, lsem[p]).wait()

                @pl.when(t_cur + 1 < OUTER)
                def _():
                    rn = row0 + (t_cur + 1) * RCHUNK
                    pltpu.async_copy(gall_hbm.at[c, pl.ds(rn, RCHUNK)],
                                     gidx2[1 - p], lsem[1 - p])
                    pltpu.async_copy(sall_hbm.at[0, pl.ds(rn, RCHUNK)],
                                     sidx2[1 - p], lsem[1 - p])

                def passes(q, carry2):
                    j0 = q * NBUF
                    for b in range(NBUF):
                        @pl.when(q > 0)
                        def _():
                            pltpu.make_async_copy(
                                rows[b], acc.at[si.at[0]], ssem[b]).wait()
                        pltpu.async_copy(
                            tab_hbm.at[gi.at[j0 + b]], rows[b], gsem[b])
                    for b in range(NBUF):
                        pltpu.make_async_copy(
                            tab_hbm.at[gi.at[0]], rows[b], gsem[b]).wait()
                        pltpu.async_copy(rows[b], acc.at[si.at[j0 + b]],
                                         ssem[b], add=True)
                    return carry2

                lax.fori_loop(0, RCHUNK // NBUF, passes, carry)
                for b in range(NBUF):
                    pltpu.make_async_copy(
                        rows[b], acc.at[si.at[0]], ssem[b]).wait()
            return carry

        lax.fori_loop(0, OUTER // 2, outer2, 0)
        plsc.subcore_barrier()
        pltpu.sync_copy(acc.at[pl.ds(acc0, APT)],
                        out_hbm.at[c, pl.ds(acc0, APT)])


def _make_combine(side):
    lo = 0 if side == 0 else TPS

    def body(dp_ref, s0_ref, s1_ref, f_ref, o_ref):
        dp = dp_ref[0]
        deg = jnp.sum(dp[lo:lo + TPS, :], axis=0)
        inv = 1.0 / (jnp.sqrt(deg) + EPS)
        sv = jnp.concatenate([s0_ref[0], s1_ref[0]], axis=1)
        o_ref[...] = 0.5 * (f_ref[...] + inv[:, None] * sv)

    return pl.pallas_call(
        body,
        grid=(NB,),
        in_specs=[
            pl.BlockSpec((1, NW, NBLK), lambda i: (i, 0, 0)),
            pl.BlockSpec((1, NBLK, 32), lambda i: (0, i, 0)),
            pl.BlockSpec((1, NBLK, 32), lambda i: (1, i, 0)),
            pl.BlockSpec((NBLK, D), lambda i: (i, 0)),
        ],
        out_specs=pl.BlockSpec((NBLK, D), lambda i: (i, 0)),
        out_shape=jax.ShapeDtypeStruct((N, D), jnp.float32),
    )


_combine_a = _make_combine(0)
_combine_b = _make_combine(1)


@jax.jit
def kernel(a_feature, b_feature, edge_index):
    ei = edge_index.astype(jnp.int32)

    zeros_hist = jnp.zeros((N,), jnp.float32)
    deg_flat, src_all, dst_all = _deg_kernel(ei.reshape(2 * E), zeros_hist)
    deg_parts = deg_flat.reshape(NB, NW, NBLK)

    tab_a, tab_b = _scale_call(deg_parts, a_feature, b_feature)

    zeros_feat = jnp.zeros((APT, 32), jnp.float32)
    src_v = src_all.reshape(2, ROWS2, CH)
    dst_v = dst_all.reshape(2, ROWS2, CH)
    s_a = _spmm_phase(tab_b.reshape(2 * N, 32), dst_v, src_v, zeros_feat)
    out_a = _combine_a(deg_parts, s_a, s_a, a_feature)
    s_b = _spmm_phase(tab_a.reshape(2 * N, 32), src_v, dst_v, zeros_feat)
    out_b = _combine_b(deg_parts, s_b, s_b, b_feature)
    return out_a, out_b
